# Initial kernel scaffold; baseline (speedup 1.0000x reference)
import jax
import jax.numpy as jnp
from jax.experimental import pallas as pl


def kernel(x):
    raise NotImplementedError("write your pallas kernel here")


# speedup vs baseline: 18.3671x; 18.3671x over previous
import functools

import jax
import jax.numpy as jnp
from jax.experimental import pallas as pl

_K = 8


def _topk_body(x_ref, idx_ref, *, n):
    xb = x_ref[...]
    iota = jax.lax.broadcasted_iota(jnp.int32, xb.shape, 1)
    cols = []
    for _ in range(_K):
        m = jnp.max(xb, axis=1, keepdims=True)
        cand = jnp.where(xb == m, iota, n)
        a = jnp.min(cand, axis=1, keepdims=True)
        cols.append(a)
        xb = jnp.where(iota == a, -jnp.inf, xb)
    idx_ref[...] = jnp.concatenate(cols, axis=1)


def _sort_body(i_ref, o_ref, *, b):
    a = i_ref[...]
    m = a.shape[1]
    iota0 = jax.lax.broadcasted_iota(jnp.int32, a.shape, 0)
    k = 2
    while k <= b:
        j = k // 2
        while j >= 1:
            g = b // (2 * j)
            a4 = a.reshape(g, 2, j, m)
            ap = jnp.concatenate([a4[:, 1:2], a4[:, 0:1]], axis=1).reshape(b, m)
            up = (iota0 & k) == 0
            low = (iota0 & j) == 0
            take_min = up == low
            a = jnp.where(take_min, jnp.minimum(a, ap), jnp.maximum(a, ap))
            j //= 2
        k *= 2
    o_ref[...] = a


def _gather_body(x_ref, idx_ref, o_ref):
    xb = x_ref[...]
    ib = idx_ref[...]
    iota = jax.lax.broadcasted_iota(jnp.int32, xb.shape, 1)
    cols = []
    for j in range(_K):
        sel = ib[:, j:j + 1]
        v = jnp.sum(jnp.where(iota == sel, xb, 0.0), axis=1, keepdims=True)
        cols.append(v)
    o_ref[...] = jnp.concatenate(cols, axis=1)


def kernel(x):
    b, c, n = x.shape
    bc = b * c
    x2 = x.reshape(bc, n)
    r = min(256, bc)

    idx = pl.pallas_call(
        functools.partial(_topk_body, n=n),
        grid=(bc // r,),
        in_specs=[pl.BlockSpec((r, n), lambda i: (i, 0))],
        out_specs=pl.BlockSpec((r, _K), lambda i: (i, 0)),
        out_shape=jax.ShapeDtypeStruct((bc, _K), jnp.int32),
    )(x2)

    idx_b = idx.reshape(b, c * _K)
    idxes = pl.pallas_call(
        functools.partial(_sort_body, b=b),
        in_specs=[pl.BlockSpec((b, c * _K), lambda: (0, 0))],
        out_specs=pl.BlockSpec((b, c * _K), lambda: (0, 0)),
        out_shape=jax.ShapeDtypeStruct((b, c * _K), jnp.int32),
    )(idx_b)
    idxes2 = idxes.reshape(bc, _K)

    out = pl.pallas_call(
        _gather_body,
        grid=(bc // r,),
        in_specs=[
            pl.BlockSpec((r, n), lambda i: (i, 0)),
            pl.BlockSpec((r, _K), lambda i: (i, 0)),
        ],
        out_specs=pl.BlockSpec((r, _K), lambda i: (i, 0)),
        out_shape=jax.ShapeDtypeStruct((bc, _K), jnp.float32),
    )(x2, idxes2)

    return out.reshape(b, c, _K)
